# baseline (speedup 1.0000x reference)
import functools

import jax
import jax.numpy as jnp
from jax import lax
from jax.experimental import pallas as pl
from jax.experimental.pallas import tpu as pltpu
from jax.experimental.pallas import tpu_sc as plsc

_T = 200
_B = 16384
_D = 32
_UB = 512
_NG = 4
_NO = 2
_PL = (_UB // 128) * 1024


def _gather_fn(v):
    info = plsc.get_sparse_core_info()
    nc, ns = info.num_cores, info.num_subcores
    nw = nc * ns
    nunits = _T * (_B // _UB)
    per_w = nunits // nw
    ppt = _B // _UB
    assert nunits % nw == 0 and per_w % _NG == 0

    mesh = plsc.VectorSubcoreMesh(core_axis_name="c", subcore_axis_name="s")

    @functools.partial(
        pl.kernel,
        out_type=jax.ShapeDtypeStruct((_T, _D // 8, (_B // 128) * 1024),
                                      jnp.float32),
        mesh=mesh,
        scratch_types=[
            pltpu.VMEM((_NG, _UB), jnp.int32),
            pltpu.VMEM((_NG, _UB, _D), jnp.float32),
            pltpu.VMEM((_NO, _D // 8, _PL), jnp.float32),
            [pltpu.SemaphoreType.DMA] * _NG,
            [pltpu.SemaphoreType.DMA] * _NG,
            [pltpu.SemaphoreType.DMA] * _NO,
        ],
        compiler_params=pltpu.CompilerParams(use_tc_tiling_on_sc=False,
                                             needs_layout_passes=False),
    )
    def run(idx_hbm, table_hbm, out_hbm, idx_v, rows_v, tile_v,
            isems, gsems, osems):
        wid = lax.axis_index("s") * nc + lax.axis_index("c")
        u0 = wid * per_w
        iota = lax.iota(jnp.int32, 16)

        rots = [(iota + i) & 15 for i in range(16)]
        wps = [(r >> 3) * _PL + (r & 7) * 128 + iota for r in rots]

        def unit_tp(u):
            t = u // ppt
            p = u - t * ppt
            return t, p

        def start_idx(s, u):
            t, p = unit_tp(u)
            off = pl.multiple_of(t * _B + p * _UB, 8)
            pltpu.async_copy(idx_hbm.at[pl.ds(off, _UB)], idx_v.at[s],
                             isems[s])

        def wait_idx(s):
            pltpu.make_async_copy(idx_hbm.at[pl.ds(0, _UB)], idx_v.at[s],
                                  isems[s]).wait()

        def start_gather(s):
            pltpu.async_copy(table_hbm.at[idx_v.at[s]], rows_v.at[s],
                             gsems[s])

        def wait_gather(s):
            pltpu.make_async_copy(table_hbm.at[idx_v.at[s]], rows_v.at[s],
                                  gsems[s]).wait()

        def start_out(s, u):
            t, p = unit_tp(u)
            pltpu.async_copy(tile_v.at[s],
                             out_hbm.at[t, :, pl.ds(p * _PL, _PL)],
                             osems[s])

        def wait_out(s):
            pltpu.make_async_copy(tile_v.at[s],
                                  out_hbm.at[0, :, pl.ds(0, _PL)],
                                  osems[s]).wait()

        def transpose(rs, ts):
            rows = rows_v.at[rs]
            tiles = tile_v.at[ts]

            def tbody(bg, carry):
                brow = iota + bg * 16
                wb0 = (bg >> 3) * 1024 + (bg & 7) * 16
                for h in range(2):
                    wbase = wb0 + h * 2 * _PL
                    for i in range(16):
                        val = plsc.load_gather(rows,
                                               [brow, rots[i] + h * 16])
                        w = wps[i] + wbase
                        plsc.store_scatter(
                            tiles, [w // _PL, lax.rem(w, _PL)], val)
                return carry

            lax.fori_loop(0, _UB // 16, tbody, 0)

        for s in range(_NG):
            start_idx(s, u0 + s)
        for s in range(3):
            wait_idx(s)
            start_gather(s)

        def body(k4, carry):
            for sv in range(_NG):
                k = k4 * _NG + sv
                ts = sv % _NO
                wait_gather(sv)

                @pl.when(k + 3 < per_w)
                def _():
                    s3 = (sv + 3) % _NG
                    wait_idx(s3)
                    start_gather(s3)

                @pl.when(k >= _NO)
                def _():
                    wait_out(ts)
                transpose(sv, ts)
                start_out(ts, u0 + k)

                @pl.when(k + _NG < per_w)
                def _():
                    start_idx(sv, u0 + k + _NG)
            return carry

        lax.fori_loop(0, per_w // _NG, body, 0)
        for s in range(_NO):
            wait_out(s)

    return run


def kernel(input, weight):
    idx_flat = jnp.transpose(input).reshape(-1).astype(jnp.int32)
    out3d = _gather_fn(weight.shape[0])(idx_flat, weight)
    out5d = out3d.reshape(_T, _D // 8, _B // 128, 8, 128)
    return jnp.transpose(out5d, (2, 4, 0, 1, 3)).reshape(_B, _T, _D)


# speedup vs baseline: 1.0925x; 1.0925x over previous
import functools

import jax
import jax.numpy as jnp
from jax import lax
from jax.experimental import pallas as pl
from jax.experimental.pallas import tpu as pltpu
from jax.experimental.pallas import tpu_sc as plsc

_T = 200
_B = 16384
_D = 32
_UB = 512
_NG = 4
_NO = 2
_PL = (_UB // 128) * 1024


def _gather_fn(v):
    info = plsc.get_sparse_core_info()
    nc, ns = info.num_cores, info.num_subcores
    nw = nc * ns
    nunits = _T * (_B // _UB)
    per_w = nunits // nw
    ppt = _B // _UB
    assert nunits % nw == 0 and per_w % _NG == 0

    mesh = plsc.VectorSubcoreMesh(core_axis_name="c", subcore_axis_name="s")

    @functools.partial(
        pl.kernel,
        out_type=jax.ShapeDtypeStruct((_T, _D // 8, (_B // 128) * 1024),
                                      jnp.float32),
        mesh=mesh,
        scratch_types=[
            pltpu.VMEM((_NG, _UB), jnp.int32),
            pltpu.VMEM((_NG, _UB, _D), jnp.float32),
            pltpu.VMEM((_NO, _D // 8, _PL), jnp.float32),
            [pltpu.SemaphoreType.DMA] * _NG,
            [pltpu.SemaphoreType.DMA] * _NG,
            [pltpu.SemaphoreType.DMA] * _NO,
        ],
        compiler_params=pltpu.CompilerParams(use_tc_tiling_on_sc=False,
                                             needs_layout_passes=False),
    )
    def run(idx_hbm, table_hbm, out_hbm, idx_v, rows_v, tile_v,
            isems, gsems, osems):
        wid = lax.axis_index("s") * nc + lax.axis_index("c")
        u0 = wid * per_w
        iota = lax.iota(jnp.int32, 16)

        rots = [(iota + i) & 15 for i in range(16)]
        wps = [(r >> 3) * _PL + (r & 7) * 128 + iota for r in rots]

        def unit_tp(u):
            t = u // ppt
            p = u - t * ppt
            return t, p

        def start_idx(s, u):
            t, p = unit_tp(u)
            off = pl.multiple_of(t * _B + p * _UB, 8)
            pltpu.async_copy(idx_hbm.at[pl.ds(off, _UB)], idx_v.at[s],
                             isems[s])

        def wait_idx(s):
            pltpu.make_async_copy(idx_hbm.at[pl.ds(0, _UB)], idx_v.at[s],
                                  isems[s]).wait()

        def start_gather(s):
            pltpu.async_copy(table_hbm.at[idx_v.at[s]], rows_v.at[s],
                             gsems[s])

        def wait_gather(s):
            pltpu.make_async_copy(table_hbm.at[idx_v.at[s]], rows_v.at[s],
                                  gsems[s]).wait()

        def start_out(s, u):
            t, p = unit_tp(u)
            pltpu.async_copy(tile_v.at[s],
                             out_hbm.at[t, :, pl.ds(p * _PL, _PL)],
                             osems[s])

        def wait_out(s):
            pltpu.make_async_copy(tile_v.at[s],
                                  out_hbm.at[0, :, pl.ds(0, _PL)],
                                  osems[s]).wait()

        def transpose(rs, ts):
            rows = rows_v.at[rs]
            tiles = tile_v.at[ts]

            def tbody(bg, carry):
                brow = iota + bg * 16
                wb0 = (bg >> 3) * 1024 + (bg & 7) * 16
                for h in range(2):
                    wbase = wb0 + h * 2 * _PL
                    for i in range(16):
                        val = plsc.load_gather(rows,
                                               [brow, rots[i] + h * 16])
                        w = wps[i] + wbase
                        plsc.store_scatter(
                            tiles, [w // _PL, lax.rem(w, _PL)], val)
                return carry

            lax.fori_loop(0, _UB // 16, tbody, 0)

        for s in range(_NG):
            start_idx(s, u0 + s)
        for s in range(2):
            wait_idx(s)
            start_gather(s)

        def body(k4, carry):
            for sv in range(_NG):
                k = k4 * _NG + sv
                ts = sv % _NO
                wait_gather(sv)

                @pl.when(k + 2 < per_w)
                def _():
                    s2 = (sv + 2) % _NG
                    wait_idx(s2)
                    start_gather(s2)

                @pl.when(k >= _NO)
                def _():
                    wait_out(ts)
                transpose(sv, ts)
                start_out(ts, u0 + k)

                @pl.when(k + _NG < per_w)
                def _():
                    start_idx(sv, u0 + k + _NG)
            return carry

        lax.fori_loop(0, per_w // _NG, body, 0)
        for s in range(_NO):
            wait_out(s)

    return run


def kernel(input, weight):
    idx_flat = jnp.transpose(input).reshape(-1).astype(jnp.int32)
    out3d = _gather_fn(weight.shape[0])(idx_flat, weight)
    out5d = out3d.reshape(_T, _D // 8, _B // 128, 8, 128)
    return jnp.transpose(out5d, (2, 4, 0, 1, 3)).reshape(_B, _T, _D)


# speedup vs baseline: 1.0934x; 1.0008x over previous
import functools

import jax
import jax.numpy as jnp
from jax import lax
from jax.experimental import pallas as pl
from jax.experimental.pallas import tpu as pltpu
from jax.experimental.pallas import tpu_sc as plsc

_T = 200
_B = 16384
_D = 32
_UB = 512
_NG = 4
_NO = 2
_PL = (_UB // 128) * 1024


def _gather_fn(v):
    info = plsc.get_sparse_core_info()
    nc, ns = info.num_cores, info.num_subcores
    nw = nc * ns
    nunits = _T * (_B // _UB)
    per_w = nunits // nw
    ppt = _B // _UB
    assert nunits % nw == 0 and per_w % _NG == 0

    mesh = plsc.VectorSubcoreMesh(core_axis_name="c", subcore_axis_name="s")

    @functools.partial(
        pl.kernel,
        out_type=jax.ShapeDtypeStruct((_T, _D // 8, (_B // 128) * 1024),
                                      jnp.float32),
        mesh=mesh,
        scratch_types=[
            pltpu.VMEM((_NG, _UB), jnp.int32),
            pltpu.VMEM((_NG, _UB, _D), jnp.float32),
            pltpu.VMEM((_NO, _D // 8, _PL), jnp.float32),
            [pltpu.SemaphoreType.DMA] * _NG,
            [pltpu.SemaphoreType.DMA] * _NG,
            [pltpu.SemaphoreType.DMA] * _NO,
        ],
        compiler_params=pltpu.CompilerParams(use_tc_tiling_on_sc=False,
                                             needs_layout_passes=False),
    )
    def run(idx_hbm, table_hbm, out_hbm, idx_v, rows_v, tile_v,
            isems, gsems, osems):
        wid = lax.axis_index("s") * nc + lax.axis_index("c")
        u0 = wid * per_w
        iota = lax.iota(jnp.int32, 16)

        rots = [(iota + i) & 15 for i in range(16)]
        plane = [r >> 3 for r in rots]
        inpl = [(r & 7) * 128 + iota for r in rots]

        def unit_tp(u):
            t = u // ppt
            p = u - t * ppt
            return t, p

        def start_idx(s, u):
            t, p = unit_tp(u)
            off = pl.multiple_of(t * _B + p * _UB, 8)
            pltpu.async_copy(idx_hbm.at[pl.ds(off, _UB)], idx_v.at[s],
                             isems[s])

        def wait_idx(s):
            pltpu.make_async_copy(idx_hbm.at[pl.ds(0, _UB)], idx_v.at[s],
                                  isems[s]).wait()

        def start_gather(s):
            pltpu.async_copy(table_hbm.at[idx_v.at[s]], rows_v.at[s],
                             gsems[s])

        def wait_gather(s):
            pltpu.make_async_copy(table_hbm.at[idx_v.at[s]], rows_v.at[s],
                                  gsems[s]).wait()

        def start_out(s, u):
            t, p = unit_tp(u)
            pltpu.async_copy(tile_v.at[s],
                             out_hbm.at[t, :, pl.ds(p * _PL, _PL)],
                             osems[s])

        def wait_out(s):
            pltpu.make_async_copy(tile_v.at[s],
                                  out_hbm.at[0, :, pl.ds(0, _PL)],
                                  osems[s]).wait()

        def transpose(rs, ts):
            rows = rows_v.at[rs]
            tiles = tile_v.at[ts]

            def tbody(bg, carry):
                brow = iota + bg * 16
                wbv = (iota * 0) + ((bg >> 3) * 1024 + (bg & 7) * 16)
                for h in range(2):
                    for i in range(16):
                        val = plsc.load_gather(rows,
                                               [brow, rots[i] + h * 16])
                        plsc.store_scatter(
                            tiles, [plane[i] + 2 * h, inpl[i] + wbv], val)
                return carry

            lax.fori_loop(0, _UB // 16, tbody, 0)

        for s in range(_NG):
            start_idx(s, u0 + s)
        for s in range(2):
            wait_idx(s)
            start_gather(s)

        def body(k4, carry):
            for sv in range(_NG):
                k = k4 * _NG + sv
                ts = sv % _NO
                wait_gather(sv)

                @pl.when(k + 2 < per_w)
                def _():
                    s2 = (sv + 2) % _NG
                    wait_idx(s2)
                    start_gather(s2)

                @pl.when(k >= _NO)
                def _():
                    wait_out(ts)
                transpose(sv, ts)
                start_out(ts, u0 + k)

                @pl.when(k + _NG < per_w)
                def _():
                    start_idx(sv, u0 + k + _NG)
            return carry

        lax.fori_loop(0, per_w // _NG, body, 0)
        for s in range(_NO):
            wait_out(s)

    return run


def kernel(input, weight):
    idx_flat = jnp.transpose(input).reshape(-1).astype(jnp.int32)
    out3d = _gather_fn(weight.shape[0])(idx_flat, weight)
    out5d = out3d.reshape(_T, _D // 8, _B // 128, 8, 128)
    return jnp.transpose(out5d, (2, 4, 0, 1, 3)).reshape(_B, _T, _D)
